# baseline (speedup 1.0000x reference)
import functools

import jax
import jax.numpy as jnp
from jax import lax
from jax.experimental import pallas as pl
from jax.experimental.pallas import tpu as pltpu
from jax.experimental.pallas import tpu_sc as plsc

N = 10000
NP = 10240
E = 320000
EP = 327680
D = 128
FH = D // 2
C = 16

NCORES = 2
NSUB = 16
NW = NCORES * NSUB
CHUNK = 128
NCHUNKS = EP // CHUNK
ROWS_PER_TILE = NP // NSUB
NBUF = 4

_MESH = plsc.VectorSubcoreMesh(core_axis_name="c", subcore_axis_name="s")


_DEG_CPW = NCHUNKS // NW


@functools.partial(
    pl.kernel,
    mesh=_MESH,
    out_type=jax.ShapeDtypeStruct((NCORES, NP), jnp.float32),
    scratch_types=[
        pltpu.VMEM((_DEG_CPW, CHUNK), jnp.int32),
        pltpu.VMEM((_DEG_CPW, CHUNK), jnp.float32),
        pltpu.VMEM((CHUNK,), jnp.float32),
        pltpu.VMEM_SHARED((NP,), jnp.float32),
        pltpu.SemaphoreType.DMA,
    ],
)
def _deg_kernel(dst_hbm, ew_hbm, out_hbm, dst_v, ew_v, buf_v, acc_sh, sem):
    c = lax.axis_index("c")
    s = lax.axis_index("s")
    w = c * NSUB + s
    zero = jnp.zeros((16,), jnp.float32)

    def zbuf(i, carry):
        buf_v[pl.ds(i * 16, 16)] = zero
        return carry
    lax.fori_loop(0, CHUNK // 16, zbuf, 0)

    def zacc(k, carry):
        pltpu.sync_copy(buf_v, acc_sh.at[pl.ds(s * ROWS_PER_TILE + k * CHUNK, CHUNK)])
        return carry
    lax.fori_loop(0, ROWS_PER_TILE // CHUNK, zacc, 0)

    pltpu.sync_copy(dst_hbm.at[pl.ds(w * _DEG_CPW, _DEG_CPW)], dst_v)
    pltpu.sync_copy(ew_hbm.at[pl.ds(w * _DEG_CPW, _DEG_CPW)], ew_v)
    plsc.subcore_barrier()

    K = 8

    def body(g, carry):
        for u in range(K):
            j = g * K + u
            pltpu.async_copy(ew_v.at[j], acc_sh.at[dst_v.at[j]], sem, add=True)
        for u in range(K):
            pltpu.make_async_copy(ew_v.at[0], acc_sh.at[dst_v.at[0]], sem).wait()
        return carry
    lax.fori_loop(0, _DEG_CPW // K, body, 0)
    plsc.subcore_barrier()

    def drain(k, carry):
        sl = pl.ds(s * ROWS_PER_TILE + k * CHUNK, CHUNK)
        pltpu.sync_copy(acc_sh.at[sl], buf_v)
        pltpu.sync_copy(buf_v, out_hbm.at[c, sl])
        return carry
    lax.fori_loop(0, ROWS_PER_TILE // CHUNK, drain, 0)


def _make_scatter(F, fsplit, ch, nbuf, nphase=1):
    nv = F // 16
    nch = EP // ch
    cpw = nch // NSUB if fsplit else nch // NW
    cpp = cpw // nphase

    @functools.partial(
        pl.kernel,
        mesh=_MESH,
        compiler_params=pltpu.CompilerParams(
            use_tc_tiling_on_sc=(F % 128 == 0)),
        out_type=jax.ShapeDtypeStruct((NCORES, NP, F), jnp.float32),
        scratch_types=[
            pltpu.VMEM((cpp, ch), jnp.int32),
            pltpu.VMEM((cpp, ch), jnp.int32),
            pltpu.VMEM((cpp * ch,), jnp.float32),
            pltpu.VMEM((nbuf, ch, F), jnp.float32),
            pltpu.VMEM((nbuf, ch, F), jnp.float32),
            pltpu.VMEM_SHARED((NP, F), jnp.float32),
        ] + [pltpu.SemaphoreType.DMA] * (2 * nbuf),
    )
    def _scatter(src_hbm, dst_hbm, ew_hbm, table_hbm, out_hbm,
                 src_v, dst_v, ew_v, bufs, obufs, acc_sh, *sems):
        gs = sems[:nbuf]
        ss = sems[nbuf:]
        c = lax.axis_index("c")
        s = lax.axis_index("s")
        w = s if fsplit else c * NSUB + s
        table = table_hbm.at[c] if fsplit else table_hbm
        zero = jnp.zeros((16,), jnp.float32)

        def zrow(i, carry):
            bufs[0, i // nv, pl.ds((i % nv) * 16, 16)] = zero
            return carry
        lax.fori_loop(0, ch * nv, zrow, 0)

        def zacc(k, carry):
            pltpu.sync_copy(
                bufs.at[0], acc_sh.at[pl.ds(s * ROWS_PER_TILE + k * ch, ch)])
            return carry
        lax.fori_loop(0, ROWS_PER_TILE // ch, zacc, 0)

        plsc.subcore_barrier()

        idxc = [jnp.full((16,), u, jnp.int32) for u in range(16)]

        def scale(b, j):
            base = j * ch

            def edge_group(g, inner):
                wvec = ew_v[pl.ds(base + g * 16, 16)]
                for u in range(16):
                    splat = wvec.at[idxc[u]].get(mode="promise_in_bounds")
                    r = g * 16 + u
                    for f in range(nv):
                        sl = pl.ds(f * 16, 16)
                        obufs[b, r, sl] = bufs[b, r, sl] * splat
                return inner
            lax.fori_loop(0, ch // 16, edge_group, 0)

        for p in range(nphase):
            base_c = w * cpw + p * cpp
            pltpu.sync_copy(src_hbm.at[pl.ds(base_c, cpp)], src_v)
            pltpu.sync_copy(dst_hbm.at[pl.ds(base_c, cpp)], dst_v)
            pltpu.sync_copy(ew_hbm.at[pl.ds(base_c * ch, cpp * ch)], ew_v)

            for b in range(nbuf):
                pltpu.async_copy(table.at[src_v.at[b]], bufs.at[b], gs[b])
            for b in range(nbuf):
                pltpu.make_async_copy(
                    table.at[src_v.at[b]], bufs.at[b], gs[b]).wait()
                scale(b, b)
                jn = jnp.minimum(jnp.int32(b + nbuf), cpp - 1)
                pltpu.async_copy(table.at[src_v.at[jn]], bufs.at[b], gs[b])
                pltpu.async_copy(
                    obufs.at[b], acc_sh.at[dst_v.at[b]], ss[b], add=True)

            def body(t, carry):
                j0 = t * nbuf
                for b in range(nbuf):
                    j = j0 + b
                    pltpu.make_async_copy(
                        table.at[src_v.at[j]], bufs.at[b], gs[b]).wait()
                    pltpu.make_async_copy(
                        obufs.at[b], acc_sh.at[dst_v.at[0]], ss[b]).wait()
                    scale(b, j)
                    jn = jnp.minimum(j + nbuf, cpp - 1)
                    pltpu.async_copy(table.at[src_v.at[jn]], bufs.at[b], gs[b])
                    pltpu.async_copy(
                        obufs.at[b], acc_sh.at[dst_v.at[j]], ss[b], add=True)
                return carry
            lax.fori_loop(1, cpp // nbuf, body, 0)
            for b in range(nbuf):
                pltpu.make_async_copy(
                    table.at[src_v.at[0]], bufs.at[b], gs[b]).wait()
                pltpu.make_async_copy(
                    obufs.at[b], acc_sh.at[dst_v.at[0]], ss[b]).wait()
        plsc.subcore_barrier()

        nst = ROWS_PER_TILE // ch
        for k in range(nst):
            b = k % 2
            sl = pl.ds(s * ROWS_PER_TILE + k * ch, ch)
            if k >= 2:
                pltpu.make_async_copy(bufs.at[b], out_hbm.at[c, sl], ss[b]).wait()
            pltpu.sync_copy(acc_sh.at[sl], bufs.at[b])
            pltpu.async_copy(bufs.at[b], out_hbm.at[c, sl], ss[b])
        for k in (nst - 2, nst - 1):
            b = k % 2
            sl = pl.ds(s * ROWS_PER_TILE + k * ch, ch)
            pltpu.make_async_copy(bufs.at[b], out_hbm.at[c, sl], ss[b]).wait()

    return _scatter


CH1 = 256
_scatter_l1 = _make_scatter(FH, fsplit=True, ch=CH1, nbuf=2, nphase=4)
_scatter_l2 = _make_scatter(C, fsplit=False, ch=CHUNK, nbuf=4)


_BLK = 1280
_GRID = NP // _BLK


def _dis(degp_ref):
    return lax.rsqrt(degp_ref[0, :] + degp_ref[1, :] + 1.0)


def _prep1_body(x_ref, w_ref, degp_ref, o_ref):
    h = jnp.dot(x_ref[...], w_ref[...], preferred_element_type=jnp.float32)
    o_ref[...] = h * _dis(degp_ref)[:, None]


def _mid_body(s1_ref, h1p_ref, degp_ref, b1_ref, w2_ref, o_ref):
    dis = _dis(degp_ref)
    tot = jnp.concatenate([s1_ref[0], s1_ref[1]], axis=-1) + h1p_ref[...]
    z = jnp.maximum(tot * dis[:, None] + b1_ref[...], 0.0)
    h2 = jnp.dot(z, w2_ref[...], preferred_element_type=jnp.float32)
    o_ref[...] = h2 * dis[:, None]


def _final_body(s2_ref, h2p_ref, degp_ref, b2_ref, o_ref):
    dis = _dis(degp_ref)
    o_ref[...] = ((s2_ref[0] + s2_ref[1] + h2p_ref[...]) * dis[:, None]
                  + b2_ref[...])


_prep1 = pl.pallas_call(
    _prep1_body,
    grid=(_GRID,),
    in_specs=[
        pl.BlockSpec((_BLK, D), lambda i: (i, 0)),
        pl.BlockSpec((D, D), lambda i: (0, 0)),
        pl.BlockSpec((NCORES, _BLK), lambda i: (0, i)),
    ],
    out_specs=pl.BlockSpec((_BLK, D), lambda i: (i, 0)),
    out_shape=jax.ShapeDtypeStruct((NP, D), jnp.float32),
)

_mid = pl.pallas_call(
    _mid_body,
    grid=(_GRID,),
    in_specs=[
        pl.BlockSpec((NCORES, _BLK, FH), lambda i: (0, i, 0)),
        pl.BlockSpec((_BLK, D), lambda i: (i, 0)),
        pl.BlockSpec((NCORES, _BLK), lambda i: (0, i)),
        pl.BlockSpec((1, D), lambda i: (0, 0)),
        pl.BlockSpec((D, C), lambda i: (0, 0)),
    ],
    out_specs=pl.BlockSpec((_BLK, C), lambda i: (i, 0)),
    out_shape=jax.ShapeDtypeStruct((NP, C), jnp.float32),
)

_final = pl.pallas_call(
    _final_body,
    grid=(_GRID,),
    in_specs=[
        pl.BlockSpec((NCORES, _BLK, C), lambda i: (0, i, 0)),
        pl.BlockSpec((_BLK, C), lambda i: (i, 0)),
        pl.BlockSpec((NCORES, _BLK), lambda i: (0, i)),
        pl.BlockSpec((1, C), lambda i: (0, 0)),
    ],
    out_specs=pl.BlockSpec((_BLK, C), lambda i: (i, 0)),
    out_shape=jax.ShapeDtypeStruct((NP, C), jnp.float32),
)


def kernel(x, edge_index, edge_weight, W1, b1, W2, b2):
    f32 = jnp.float32
    src = edge_index[0]
    dst = edge_index[1]
    padn = EP - E
    pad_idx = (N + (jnp.arange(padn, dtype=jnp.int32) % (NP - N))).astype(jnp.int32)
    src_f = jnp.concatenate([src, pad_idx])
    dst_f = jnp.concatenate([dst, pad_idx])
    src_p = src_f.reshape(NCHUNKS, CHUNK)
    dst_p = dst_f.reshape(NCHUNKS, CHUNK)
    src_p1 = src_f.reshape(EP // CH1, CH1)
    dst_p1 = dst_f.reshape(EP // CH1, CH1)
    ew_flat = jnp.concatenate([edge_weight, jnp.zeros((padn,), f32)])
    ew2 = ew_flat.reshape(NCHUNKS, CHUNK)
    x_p = jnp.pad(x, ((0, NP - N), (0, 0)))

    degp = _deg_kernel(dst_p, ew2)
    h1p = _prep1(x_p, W1, degp)
    h1p_halves = h1p.reshape(NP, NCORES, FH).transpose(1, 0, 2)
    s1p = _scatter_l1(src_p1, dst_p1, ew_flat, h1p_halves)
    h2p = _mid(s1p, h1p, degp, b1.reshape(1, D), W2)
    s2p = _scatter_l2(src_p, dst_p, ew_flat, h2p)
    outp = _final(s2p, h2p, degp, b2.reshape(1, C))
    return outp[:N]


# speedup vs baseline: 1.0147x; 1.0147x over previous
import functools

import jax
import jax.numpy as jnp
from jax import lax
from jax.experimental import pallas as pl
from jax.experimental.pallas import tpu as pltpu
from jax.experimental.pallas import tpu_sc as plsc

N = 10000
NP = 10240
E = 320000
EP = 327680
D = 128
FH = D // 2
C = 16

NCORES = 2
NSUB = 16
NW = NCORES * NSUB
CHUNK = 128
NCHUNKS = EP // CHUNK
ROWS_PER_TILE = NP // NSUB
NBUF = 4

_MESH = plsc.VectorSubcoreMesh(core_axis_name="c", subcore_axis_name="s")


_DEG_CPW = NCHUNKS // NW


@functools.partial(
    pl.kernel,
    mesh=_MESH,
    out_type=jax.ShapeDtypeStruct((NCORES, NP), jnp.float32),
    scratch_types=[
        pltpu.VMEM((_DEG_CPW, CHUNK), jnp.int32),
        pltpu.VMEM((_DEG_CPW, CHUNK), jnp.float32),
        pltpu.VMEM((CHUNK,), jnp.float32),
        pltpu.VMEM_SHARED((NP,), jnp.float32),
        pltpu.SemaphoreType.DMA,
    ],
)
def _deg_kernel(dst_hbm, ew_hbm, out_hbm, dst_v, ew_v, buf_v, acc_sh, sem):
    c = lax.axis_index("c")
    s = lax.axis_index("s")
    w = c * NSUB + s
    zero = jnp.zeros((16,), jnp.float32)

    def zbuf(i, carry):
        buf_v[pl.ds(i * 16, 16)] = zero
        return carry
    lax.fori_loop(0, CHUNK // 16, zbuf, 0)

    def zacc(k, carry):
        pltpu.sync_copy(buf_v, acc_sh.at[pl.ds(s * ROWS_PER_TILE + k * CHUNK, CHUNK)])
        return carry
    lax.fori_loop(0, ROWS_PER_TILE // CHUNK, zacc, 0)

    pltpu.sync_copy(dst_hbm.at[pl.ds(w * _DEG_CPW, _DEG_CPW)], dst_v)
    pltpu.sync_copy(ew_hbm.at[pl.ds(w * _DEG_CPW, _DEG_CPW)], ew_v)
    plsc.subcore_barrier()

    K = 8

    def body(g, carry):
        for u in range(K):
            j = g * K + u
            pltpu.async_copy(ew_v.at[j], acc_sh.at[dst_v.at[j]], sem, add=True)
        for u in range(K):
            pltpu.make_async_copy(ew_v.at[0], acc_sh.at[dst_v.at[0]], sem).wait()
        return carry
    lax.fori_loop(0, _DEG_CPW // K, body, 0)
    plsc.subcore_barrier()

    def drain(k, carry):
        sl = pl.ds(s * ROWS_PER_TILE + k * CHUNK, CHUNK)
        pltpu.sync_copy(acc_sh.at[sl], buf_v)
        pltpu.sync_copy(buf_v, out_hbm.at[c, sl])
        return carry
    lax.fori_loop(0, ROWS_PER_TILE // CHUNK, drain, 0)


def _make_scatter(F, fsplit, ch, nbuf, nphase=1):
    nv = F // 16
    nch = EP // ch
    cpw = nch // NSUB if fsplit else nch // NW
    cpp = cpw // nphase

    @functools.partial(
        pl.kernel,
        mesh=_MESH,
        compiler_params=pltpu.CompilerParams(
            use_tc_tiling_on_sc=(F % 128 == 0)),
        out_type=jax.ShapeDtypeStruct((NCORES, NP, F), jnp.float32),
        scratch_types=[
            pltpu.VMEM((cpp, ch), jnp.int32),
            pltpu.VMEM((cpp, ch), jnp.int32),
            pltpu.VMEM((cpp * ch,), jnp.float32),
            pltpu.VMEM((nbuf, ch, F), jnp.float32),
            pltpu.VMEM((nbuf, ch, F), jnp.float32),
            pltpu.VMEM_SHARED((NP, F), jnp.float32),
        ] + [pltpu.SemaphoreType.DMA] * (2 * nbuf),
    )
    def _scatter(src_hbm, dst_hbm, ew_hbm, table_hbm, out_hbm,
                 src_v, dst_v, ew_v, bufs, obufs, acc_sh, *sems):
        gs = sems[:nbuf]
        ss = sems[nbuf:]
        c = lax.axis_index("c")
        s = lax.axis_index("s")
        w = s if fsplit else c * NSUB + s
        table = table_hbm.at[c] if fsplit else table_hbm
        zero = jnp.zeros((16,), jnp.float32)

        def zrow(i, carry):
            bufs[0, i // nv, pl.ds((i % nv) * 16, 16)] = zero
            return carry
        lax.fori_loop(0, ch * nv, zrow, 0)

        def zacc(k, carry):
            pltpu.sync_copy(
                bufs.at[0], acc_sh.at[pl.ds(s * ROWS_PER_TILE + k * ch, ch)])
            return carry
        lax.fori_loop(0, ROWS_PER_TILE // ch, zacc, 0)

        plsc.subcore_barrier()

        idxc = [jnp.full((16,), u, jnp.int32) for u in range(16)]

        def scale(b, j):
            base = j * ch

            def edge_group(g, inner):
                wvec = ew_v[pl.ds(base + g * 16, 16)]
                for u in range(16):
                    splat = wvec.at[idxc[u]].get(mode="promise_in_bounds")
                    r = g * 16 + u
                    for f in range(nv):
                        sl = pl.ds(f * 16, 16)
                        obufs[b, r, sl] = bufs[b, r, sl] * splat
                return inner
            lax.fori_loop(0, ch // 16, edge_group, 0)

        for p in range(nphase):
            base_c = w * cpw + p * cpp
            pltpu.sync_copy(src_hbm.at[pl.ds(base_c, cpp)], src_v)
            pltpu.sync_copy(dst_hbm.at[pl.ds(base_c, cpp)], dst_v)
            pltpu.sync_copy(ew_hbm.at[pl.ds(base_c * ch, cpp * ch)], ew_v)

            for b in range(nbuf):
                pltpu.async_copy(table.at[src_v.at[b]], bufs.at[b], gs[b])
            for b in range(nbuf):
                pltpu.make_async_copy(
                    table.at[src_v.at[b]], bufs.at[b], gs[b]).wait()
                scale(b, b)
                jn = jnp.minimum(jnp.int32(b + nbuf), cpp - 1)
                pltpu.async_copy(table.at[src_v.at[jn]], bufs.at[b], gs[b])
                pltpu.async_copy(
                    obufs.at[b], acc_sh.at[dst_v.at[b]], ss[b], add=True)

            def body(t, carry):
                j0 = t * nbuf
                for b in range(nbuf):
                    j = j0 + b
                    pltpu.make_async_copy(
                        table.at[src_v.at[j]], bufs.at[b], gs[b]).wait()
                    pltpu.make_async_copy(
                        obufs.at[b], acc_sh.at[dst_v.at[0]], ss[b]).wait()
                    scale(b, j)
                    jn = jnp.minimum(j + nbuf, cpp - 1)
                    pltpu.async_copy(table.at[src_v.at[jn]], bufs.at[b], gs[b])
                    pltpu.async_copy(
                        obufs.at[b], acc_sh.at[dst_v.at[j]], ss[b], add=True)
                return carry
            lax.fori_loop(1, cpp // nbuf, body, 0)
            for b in range(nbuf):
                pltpu.make_async_copy(
                    table.at[src_v.at[0]], bufs.at[b], gs[b]).wait()
                pltpu.make_async_copy(
                    obufs.at[b], acc_sh.at[dst_v.at[0]], ss[b]).wait()
        plsc.subcore_barrier()

        nst = ROWS_PER_TILE // ch
        for k in range(nst):
            b = k % 2
            sl = pl.ds(s * ROWS_PER_TILE + k * ch, ch)
            if k >= 2:
                pltpu.make_async_copy(bufs.at[b], out_hbm.at[c, sl], ss[b]).wait()
            pltpu.sync_copy(acc_sh.at[sl], bufs.at[b])
            pltpu.async_copy(bufs.at[b], out_hbm.at[c, sl], ss[b])
        for k in (nst - 2, nst - 1):
            b = k % 2
            sl = pl.ds(s * ROWS_PER_TILE + k * ch, ch)
            pltpu.make_async_copy(bufs.at[b], out_hbm.at[c, sl], ss[b]).wait()

    return _scatter


CH1 = 128
_scatter_l1 = _make_scatter(FH, fsplit=True, ch=CH1, nbuf=2, nphase=2)
_scatter_l2 = _make_scatter(C, fsplit=False, ch=CHUNK, nbuf=4)


_BLK = 1280
_GRID = NP // _BLK


def _dis(degp_ref):
    return lax.rsqrt(degp_ref[0, :] + degp_ref[1, :] + 1.0)


def _prep1_body(x_ref, w_ref, degp_ref, o_ref):
    h = jnp.dot(x_ref[...], w_ref[...], preferred_element_type=jnp.float32)
    o_ref[...] = h * _dis(degp_ref)[:, None]


def _mid_body(s1_ref, h1p_ref, degp_ref, b1_ref, w2_ref, o_ref):
    dis = _dis(degp_ref)
    tot = jnp.concatenate([s1_ref[0], s1_ref[1]], axis=-1) + h1p_ref[...]
    z = jnp.maximum(tot * dis[:, None] + b1_ref[...], 0.0)
    h2 = jnp.dot(z, w2_ref[...], preferred_element_type=jnp.float32)
    o_ref[...] = h2 * dis[:, None]


def _final_body(s2_ref, h2p_ref, degp_ref, b2_ref, o_ref):
    dis = _dis(degp_ref)
    o_ref[...] = ((s2_ref[0] + s2_ref[1] + h2p_ref[...]) * dis[:, None]
                  + b2_ref[...])


_prep1 = pl.pallas_call(
    _prep1_body,
    grid=(_GRID,),
    in_specs=[
        pl.BlockSpec((_BLK, D), lambda i: (i, 0)),
        pl.BlockSpec((D, D), lambda i: (0, 0)),
        pl.BlockSpec((NCORES, _BLK), lambda i: (0, i)),
    ],
    out_specs=pl.BlockSpec((_BLK, D), lambda i: (i, 0)),
    out_shape=jax.ShapeDtypeStruct((NP, D), jnp.float32),
)

_mid = pl.pallas_call(
    _mid_body,
    grid=(_GRID,),
    in_specs=[
        pl.BlockSpec((NCORES, _BLK, FH), lambda i: (0, i, 0)),
        pl.BlockSpec((_BLK, D), lambda i: (i, 0)),
        pl.BlockSpec((NCORES, _BLK), lambda i: (0, i)),
        pl.BlockSpec((1, D), lambda i: (0, 0)),
        pl.BlockSpec((D, C), lambda i: (0, 0)),
    ],
    out_specs=pl.BlockSpec((_BLK, C), lambda i: (i, 0)),
    out_shape=jax.ShapeDtypeStruct((NP, C), jnp.float32),
)

_final = pl.pallas_call(
    _final_body,
    grid=(_GRID,),
    in_specs=[
        pl.BlockSpec((NCORES, _BLK, C), lambda i: (0, i, 0)),
        pl.BlockSpec((_BLK, C), lambda i: (i, 0)),
        pl.BlockSpec((NCORES, _BLK), lambda i: (0, i)),
        pl.BlockSpec((1, C), lambda i: (0, 0)),
    ],
    out_specs=pl.BlockSpec((_BLK, C), lambda i: (i, 0)),
    out_shape=jax.ShapeDtypeStruct((NP, C), jnp.float32),
)


def kernel(x, edge_index, edge_weight, W1, b1, W2, b2):
    f32 = jnp.float32
    src = edge_index[0]
    dst = edge_index[1]
    padn = EP - E
    pad_idx = (N + (jnp.arange(padn, dtype=jnp.int32) % (NP - N))).astype(jnp.int32)
    src_f = jnp.concatenate([src, pad_idx])
    dst_f = jnp.concatenate([dst, pad_idx])
    src_p = src_f.reshape(NCHUNKS, CHUNK)
    dst_p = dst_f.reshape(NCHUNKS, CHUNK)
    src_p1 = src_f.reshape(EP // CH1, CH1)
    dst_p1 = dst_f.reshape(EP // CH1, CH1)
    ew_flat = jnp.concatenate([edge_weight, jnp.zeros((padn,), f32)])
    ew2 = ew_flat.reshape(NCHUNKS, CHUNK)
    x_p = jnp.pad(x, ((0, NP - N), (0, 0)))

    degp = _deg_kernel(dst_p, ew2)
    h1p = _prep1(x_p, W1, degp)
    h1p_halves = h1p.reshape(NP, NCORES, FH).transpose(1, 0, 2)
    s1p = _scatter_l1(src_p1, dst_p1, ew_flat, h1p_halves)
    h2p = _mid(s1p, h1p, degp, b1.reshape(1, D), W2)
    s2p = _scatter_l2(src_p, dst_p, ew_flat, h2p)
    outp = _final(s2p, h2p, degp, b2.reshape(1, C))
    return outp[:N]
